# Initial kernel scaffold; baseline (speedup 1.0000x reference)
import jax
import jax.numpy as jnp
from jax.experimental import pallas as pl


def kernel(x, edge_index, batch, W1, b1, W2, b2, W3, b3, Wh1, bh1, Wh2, bh2):
    raise NotImplementedError("write your pallas kernel here")


# speedup vs baseline: 5.7116x; 5.7116x over previous
import functools

import jax
import jax.numpy as jnp
from jax import lax
from jax.experimental import pallas as pl
from jax.experimental.pallas import tpu as pltpu
from jax.experimental.pallas import tpu_sc as plsc

N = 10000
E = 160000
G = 64

NPAD = 10240
KPAD = 512
CHUNK = 128
EPAD = 163840
NC = 2
NS = 16
NW = NC * NS
ROWS_W = EPAD // (NW * CHUNK)
NT_ROWS = NPAD // NS
BLK = 256


def _make_deg():
    mesh = plsc.VectorSubcoreMesh(core_axis_name="c", subcore_axis_name="s")

    @functools.partial(
        pl.kernel,
        mesh=mesh,
        out_type=jax.ShapeDtypeStruct((NC, NPAD, 16), jnp.float32),
        compiler_params=pltpu.CompilerParams(use_tc_tiling_on_sc=False),
        scratch_types=[
            pltpu.VMEM((ROWS_W, CHUNK), jnp.int32),
            pltpu.VMEM((CHUNK, 16), jnp.float32),
            pltpu.VMEM_SHARED((NPAD, 16), jnp.float32),
        ],
    )
    def deg(dst_hbm, ones_hbm, zeros_hbm, out_hbm, dst_v, ones_v, acc):
        c = lax.axis_index("c")
        s = lax.axis_index("s")
        wid = s * NC + c
        pltpu.sync_copy(zeros_hbm, acc.at[pl.ds(s * NT_ROWS, NT_ROWS)])
        pltpu.sync_copy(dst_hbm.at[pl.ds(wid * ROWS_W, ROWS_W)], dst_v)
        pltpu.sync_copy(ones_hbm, ones_v)
        plsc.subcore_barrier()

        def body(j, carry):
            pltpu.sync_copy(ones_v, acc.at[dst_v.at[j]], add=True)
            return carry

        lax.fori_loop(0, ROWS_W, body, 0)
        plsc.subcore_barrier()
        pltpu.sync_copy(acc.at[pl.ds(s * NT_ROWS, NT_ROWS)],
                        out_hbm.at[c].at[pl.ds(s * NT_ROWS, NT_ROWS)])

    return deg


def _make_prop(D):
    mesh = plsc.VectorSubcoreMesh(core_axis_name="c", subcore_axis_name="s")

    @functools.partial(
        pl.kernel,
        mesh=mesh,
        out_type=jax.ShapeDtypeStruct((NC, NPAD, D), jnp.float32),
        compiler_params=pltpu.CompilerParams(use_tc_tiling_on_sc=False),
        scratch_types=[
            pltpu.VMEM((ROWS_W, CHUNK), jnp.int32),
            pltpu.VMEM((ROWS_W, CHUNK), jnp.int32),
            pltpu.VMEM((CHUNK, D), jnp.float32),
            pltpu.VMEM_SHARED((NPAD, D), jnp.float32),
            pltpu.SemaphoreType.DMA,
        ],
    )
    def prop(g_hbm, src_hbm, dst_hbm, zeros_hbm, out_hbm,
             src_v, dst_v, buf, acc, sem):
        c = lax.axis_index("c")
        s = lax.axis_index("s")
        wid = s * NC + c
        pltpu.sync_copy(zeros_hbm, acc.at[pl.ds(s * NT_ROWS, NT_ROWS)])
        base = wid * ROWS_W
        pltpu.sync_copy(src_hbm.at[pl.ds(base, ROWS_W)], src_v)
        pltpu.sync_copy(dst_hbm.at[pl.ds(base, ROWS_W)], dst_v)
        plsc.subcore_barrier()

        def body(j, carry):
            pltpu.async_copy(g_hbm.at[src_v.at[j]], buf, sem).wait()
            pltpu.sync_copy(buf, acc.at[dst_v.at[j]], add=True)
            return carry

        lax.fori_loop(0, ROWS_W, body, 0)
        plsc.subcore_barrier()
        pltpu.sync_copy(acc.at[pl.ds(s * NT_ROWS, NT_ROWS)],
                        out_hbm.at[c].at[pl.ds(s * NT_ROWS, NT_ROWS)])

    return prop


_DEG = _make_deg()
_PROP = {128: _make_prop(128), 64: _make_prop(64)}


def _kA(x_ref, w_ref, degp_ref, g_ref, dis_ref):
    deg = degp_ref[0, :, 0:1] + degp_ref[1, :, 0:1] + 1.0
    dis = lax.rsqrt(deg)
    g_ref[...] = jnp.dot(x_ref[...], w_ref[...],
                         preferred_element_type=jnp.float32) * dis
    dis_ref[...] = dis


def _tc_first(x_p, w1_p, degp):
    return pl.pallas_call(
        _kA,
        grid=(NPAD // BLK,),
        in_specs=[
            pl.BlockSpec((BLK, KPAD), lambda i: (i, 0)),
            pl.BlockSpec((KPAD, 128), lambda i: (0, 0)),
            pl.BlockSpec((NC, BLK, 16), lambda i: (0, i, 0)),
        ],
        out_specs=[
            pl.BlockSpec((BLK, 128), lambda i: (i, 0)),
            pl.BlockSpec((BLK, 1), lambda i: (i, 0)),
        ],
        out_shape=[
            jax.ShapeDtypeStruct((NPAD, 128), jnp.float32),
            jax.ShapeDtypeStruct((NPAD, 1), jnp.float32),
        ],
    )(x_p, w1_p, degp)


def _make_mid(din, dout):
    def body(p_ref, g_ref, dis_ref, b_ref, w_ref, out_ref):
        dis = dis_ref[...]
        h = jnp.maximum((p_ref[0] + p_ref[1] + g_ref[...]) * dis + b_ref[...],
                        0.0)
        out_ref[...] = jnp.dot(h, w_ref[...],
                               preferred_element_type=jnp.float32) * dis

    def call(p, g, dis, b, w):
        return pl.pallas_call(
            body,
            grid=(NPAD // BLK,),
            in_specs=[
                pl.BlockSpec((NC, BLK, din), lambda i: (0, i, 0)),
                pl.BlockSpec((BLK, din), lambda i: (i, 0)),
                pl.BlockSpec((BLK, 1), lambda i: (i, 0)),
                pl.BlockSpec((1, din), lambda i: (0, 0)),
                pl.BlockSpec((din, dout), lambda i: (0, 0)),
            ],
            out_specs=pl.BlockSpec((BLK, dout), lambda i: (i, 0)),
            out_shape=jax.ShapeDtypeStruct((NPAD, dout), jnp.float32),
        )(p, g, dis, b, w)

    return call


_MID_128_128 = _make_mid(128, 128)
_MID_128_64 = _make_mid(128, 64)


def _kC(p_ref, g_ref, dis_ref, b_ref, batch_ref, wh1_ref, bh1_ref,
        wh2_ref, bh2_ref, out_ref):
    h = jnp.maximum((p_ref[0] + p_ref[1] + g_ref[...]) * dis_ref[...]
                    + b_ref[...], 0.0)
    batch = batch_ref[...]
    neg = jnp.float32(-3.0e38)
    rows = []
    for gid in range(G):
        m = batch == gid
        rows.append(jnp.max(jnp.where(m, h, neg), axis=0, keepdims=True))
    pooled = jnp.concatenate(rows, axis=0)
    z = jnp.maximum(jnp.dot(pooled, wh1_ref[...],
                            preferred_element_type=jnp.float32)
                    + bh1_ref[...], 0.0)
    out_ref[...] = jnp.dot(z, wh2_ref[...],
                           preferred_element_type=jnp.float32) + bh2_ref[...]


def _tc_head(p, g, dis, b, batch_p, wh1, bh1, wh2, bh2):
    return pl.pallas_call(
        _kC,
        out_shape=jax.ShapeDtypeStruct((G, 1), jnp.float32),
    )(p, g, dis, b, batch_p, wh1, bh1, wh2, bh2)


def kernel(x, edge_index, batch, W1, b1, W2, b2, W3, b3, Wh1, bh1, Wh2, bh2):
    f32 = jnp.float32
    x_p = jnp.zeros((NPAD, KPAD), f32).at[:N, :397].set(x)
    w1_p = jnp.zeros((KPAD, 128), f32).at[:397].set(W1)
    src = jnp.full((EPAD,), N, jnp.int32).at[:E].set(edge_index[0])
    dst = jnp.full((EPAD,), N, jnp.int32).at[:E].set(edge_index[1])
    src2 = src.reshape(EPAD // CHUNK, CHUNK)
    dst2 = dst.reshape(EPAD // CHUNK, CHUNK)
    batch_p = jnp.full((NPAD, 1), G, jnp.int32).at[:N, 0].set(batch)
    zeros128 = jnp.zeros((NT_ROWS, 128), f32)
    zeros64 = jnp.zeros((NT_ROWS, 64), f32)
    zeros16 = jnp.zeros((NT_ROWS, 16), f32)
    ones16 = jnp.ones((CHUNK, 16), f32)

    degp = _DEG(dst2, ones16, zeros16)
    g1, dis = _tc_first(x_p, w1_p, degp)
    p1 = _PROP[128](g1, src2, dst2, zeros128)
    g2 = _MID_128_128(p1, g1, dis, b1.reshape(1, 128), W2)
    p2 = _PROP[128](g2, src2, dst2, zeros128)
    g3 = _MID_128_64(p2, g2, dis, b2.reshape(1, 128), W3)
    p3 = _PROP[64](g3, src2, dst2, zeros64)
    out = _tc_head(p3, g3, dis, b3.reshape(1, 64), batch_p,
                   Wh1, bh1.reshape(1, 32), Wh2, bh2.reshape(1, 1))
    return out[:, 0]


# speedup vs baseline: 7.0354x; 1.2318x over previous
import functools

import jax
import jax.numpy as jnp
from jax import lax
from jax.experimental import pallas as pl
from jax.experimental.pallas import tpu as pltpu
from jax.experimental.pallas import tpu_sc as plsc

N = 10000
E = 160000
G = 64

NPAD = 10240
KPAD = 512
CHUNK = 128
EPAD = 163840
NC = 2
NS = 16
NW = NC * NS
ROWS_W = EPAD // (NW * CHUNK)
NT_ROWS = NPAD // NS
BLK = 256


def _make_deg():
    mesh = plsc.VectorSubcoreMesh(core_axis_name="c", subcore_axis_name="s")

    @functools.partial(
        pl.kernel,
        mesh=mesh,
        out_type=jax.ShapeDtypeStruct((NC, NPAD, 16), jnp.float32),
        compiler_params=pltpu.CompilerParams(use_tc_tiling_on_sc=False),
        scratch_types=[
            pltpu.VMEM((ROWS_W, CHUNK), jnp.int32),
            pltpu.VMEM((CHUNK, 16), jnp.float32),
            pltpu.VMEM_SHARED((NPAD, 16), jnp.float32),
        ],
    )
    def deg(dst_hbm, ones_hbm, zeros_hbm, out_hbm, dst_v, ones_v, acc):
        c = lax.axis_index("c")
        s = lax.axis_index("s")
        wid = s * NC + c
        pltpu.sync_copy(zeros_hbm, acc.at[pl.ds(s * NT_ROWS, NT_ROWS)])
        pltpu.sync_copy(dst_hbm.at[pl.ds(wid * ROWS_W, ROWS_W)], dst_v)
        pltpu.sync_copy(ones_hbm, ones_v)
        plsc.subcore_barrier()

        def body(j, carry):
            pltpu.sync_copy(ones_v, acc.at[dst_v.at[j]], add=True)
            return carry

        lax.fori_loop(0, ROWS_W, body, 0)
        plsc.subcore_barrier()
        pltpu.sync_copy(acc.at[pl.ds(s * NT_ROWS, NT_ROWS)],
                        out_hbm.at[c].at[pl.ds(s * NT_ROWS, NT_ROWS)])

    return deg


def _make_prop(D):
    mesh = plsc.VectorSubcoreMesh(core_axis_name="c", subcore_axis_name="s")

    @functools.partial(
        pl.kernel,
        mesh=mesh,
        out_type=jax.ShapeDtypeStruct((NC, NPAD, D), jnp.float32),
        compiler_params=pltpu.CompilerParams(use_tc_tiling_on_sc=False),
        scratch_types=[
            pltpu.VMEM((ROWS_W, CHUNK), jnp.int32),
            pltpu.VMEM((ROWS_W, CHUNK), jnp.int32),
            pltpu.VMEM((CHUNK, D), jnp.float32),
            pltpu.VMEM((CHUNK, D), jnp.float32),
            pltpu.VMEM_SHARED((NPAD, D), jnp.float32),
            pltpu.SemaphoreType.DMA,
            pltpu.SemaphoreType.DMA,
        ],
    )
    def prop(g_hbm, src_hbm, dst_hbm, zeros_hbm, out_hbm,
             src_v, dst_v, buf_a, buf_b, acc, sem_a, sem_b):
        c = lax.axis_index("c")
        s = lax.axis_index("s")
        wid = s * NC + c
        pltpu.sync_copy(zeros_hbm, acc.at[pl.ds(s * NT_ROWS, NT_ROWS)])
        base = wid * ROWS_W
        pltpu.sync_copy(src_hbm.at[pl.ds(base, ROWS_W)], src_v)
        pltpu.sync_copy(dst_hbm.at[pl.ds(base, ROWS_W)], dst_v)
        plsc.subcore_barrier()

        pltpu.async_copy(g_hbm.at[src_v.at[0]], buf_a, sem_a)

        def body(i, carry):
            j = 2 * i
            pltpu.async_copy(g_hbm.at[src_v.at[j + 1]], buf_b, sem_b)
            pltpu.make_async_copy(g_hbm.at[src_v.at[j]], buf_a, sem_a).wait()
            pltpu.sync_copy(buf_a, acc.at[dst_v.at[j]], add=True)

            @pl.when(i + 1 < ROWS_W // 2)
            def _():
                pltpu.async_copy(g_hbm.at[src_v.at[j + 2]], buf_a, sem_a)

            pltpu.make_async_copy(g_hbm.at[src_v.at[j + 1]], buf_b,
                                  sem_b).wait()
            pltpu.sync_copy(buf_b, acc.at[dst_v.at[j + 1]], add=True)
            return carry

        lax.fori_loop(0, ROWS_W // 2, body, 0)
        plsc.subcore_barrier()
        pltpu.sync_copy(acc.at[pl.ds(s * NT_ROWS, NT_ROWS)],
                        out_hbm.at[c].at[pl.ds(s * NT_ROWS, NT_ROWS)])

    return prop


_DEG = _make_deg()
_PROP = {128: _make_prop(128), 64: _make_prop(64)}


def _kA(x_ref, w_ref, degp_ref, g_ref, dis_ref):
    deg = degp_ref[0, :, 0:1] + degp_ref[1, :, 0:1] + 1.0
    dis = lax.rsqrt(deg)
    g_ref[...] = jnp.dot(x_ref[...], w_ref[...],
                         preferred_element_type=jnp.float32) * dis
    dis_ref[...] = dis


def _tc_first(x, w1, degp):
    return pl.pallas_call(
        _kA,
        grid=(NPAD // BLK,),
        in_specs=[
            pl.BlockSpec((BLK, 397), lambda i: (i, 0)),
            pl.BlockSpec((397, 128), lambda i: (0, 0)),
            pl.BlockSpec((NC, BLK, 16), lambda i: (0, i, 0)),
        ],
        out_specs=[
            pl.BlockSpec((BLK, 128), lambda i: (i, 0)),
            pl.BlockSpec((BLK, 1), lambda i: (i, 0)),
        ],
        out_shape=[
            jax.ShapeDtypeStruct((NPAD, 128), jnp.float32),
            jax.ShapeDtypeStruct((NPAD, 1), jnp.float32),
        ],
    )(x, w1, degp)


def _make_mid(din, dout):
    def body(p_ref, g_ref, dis_ref, b_ref, w_ref, out_ref):
        dis = dis_ref[...]
        h = jnp.maximum((p_ref[0] + p_ref[1] + g_ref[...]) * dis + b_ref[...],
                        0.0)
        out_ref[...] = jnp.dot(h, w_ref[...],
                               preferred_element_type=jnp.float32) * dis

    def call(p, g, dis, b, w):
        return pl.pallas_call(
            body,
            grid=(NPAD // BLK,),
            in_specs=[
                pl.BlockSpec((NC, BLK, din), lambda i: (0, i, 0)),
                pl.BlockSpec((BLK, din), lambda i: (i, 0)),
                pl.BlockSpec((BLK, 1), lambda i: (i, 0)),
                pl.BlockSpec((1, din), lambda i: (0, 0)),
                pl.BlockSpec((din, dout), lambda i: (0, 0)),
            ],
            out_specs=pl.BlockSpec((BLK, dout), lambda i: (i, 0)),
            out_shape=jax.ShapeDtypeStruct((NPAD, dout), jnp.float32),
        )(p, g, dis, b, w)

    return call


_MID_128_128 = _make_mid(128, 128)
_MID_128_64 = _make_mid(128, 64)


def _kC(p_ref, g_ref, dis_ref, b_ref, batch_ref, wh1_ref, bh1_ref,
        wh2_ref, bh2_ref, out_ref):
    h = jnp.maximum((p_ref[0] + p_ref[1] + g_ref[...]) * dis_ref[...]
                    + b_ref[...], 0.0)
    batch = batch_ref[...]
    neg = jnp.float32(-3.0e38)
    rows = []
    for gid in range(G):
        m = batch == gid
        rows.append(jnp.max(jnp.where(m, h, neg), axis=0, keepdims=True))
    pooled = jnp.concatenate(rows, axis=0)
    z = jnp.maximum(jnp.dot(pooled, wh1_ref[...],
                            preferred_element_type=jnp.float32)
                    + bh1_ref[...], 0.0)
    out_ref[...] = jnp.dot(z, wh2_ref[...],
                           preferred_element_type=jnp.float32) + bh2_ref[...]


def _tc_head(p, g, dis, b, batch_p, wh1, bh1, wh2, bh2):
    return pl.pallas_call(
        _kC,
        out_shape=jax.ShapeDtypeStruct((G, 1), jnp.float32),
    )(p, g, dis, b, batch_p, wh1, bh1, wh2, bh2)


def kernel(x, edge_index, batch, W1, b1, W2, b2, W3, b3, Wh1, bh1, Wh2, bh2):
    f32 = jnp.float32
    src = jnp.full((EPAD,), N, jnp.int32).at[:E].set(edge_index[0])
    dst = jnp.full((EPAD,), N, jnp.int32).at[:E].set(edge_index[1])
    src2 = src.reshape(EPAD // CHUNK, CHUNK)
    dst2 = dst.reshape(EPAD // CHUNK, CHUNK)
    batch_p = jnp.full((NPAD, 1), G, jnp.int32).at[:N, 0].set(batch)
    zeros128 = jnp.zeros((NT_ROWS, 128), f32)
    zeros64 = jnp.zeros((NT_ROWS, 64), f32)
    zeros16 = jnp.zeros((NT_ROWS, 16), f32)
    ones16 = jnp.ones((CHUNK, 16), f32)

    degp = _DEG(dst2, ones16, zeros16)
    g1, dis = _tc_first(x, W1, degp)
    p1 = _PROP[128](g1, src2, dst2, zeros128)
    g2 = _MID_128_128(p1, g1, dis, b1.reshape(1, 128), W2)
    p2 = _PROP[128](g2, src2, dst2, zeros128)
    g3 = _MID_128_64(p2, g2, dis, b2.reshape(1, 128), W3)
    p3 = _PROP[64](g3, src2, dst2, zeros64)
    out = _tc_head(p3, g3, dis, b3.reshape(1, 64), batch_p,
                   Wh1, bh1.reshape(1, 32), Wh2, bh2.reshape(1, 1))
    return out[:, 0]
